# Initial kernel scaffold; baseline (speedup 1.0000x reference)
import jax
import jax.numpy as jnp
from jax.experimental import pallas as pl


def kernel(x, W):
    raise NotImplementedError("write your pallas kernel here")


# speedup vs baseline: 6.3944x; 6.3944x over previous
import functools

import jax
import jax.numpy as jnp
from jax import lax
from jax.experimental import pallas as pl
from jax.experimental.pallas import tpu as pltpu
from jax.experimental.pallas import tpu_sc as plsc

AA_REP = 20
MAX_PEP = 15
FILTER = 3
N_POCKETS = 34
OUT_L = AA_REP + FILTER - 1
X_COLS = 1 + MAX_PEP * AA_REP + N_POCKETS
O_COLS = N_POCKETS * OUT_L
POCKET_OFF = 1 + MAX_PEP * AA_REP

_P2J = {0: [0], 1: [1, 2], 2: [0, 1], 3: [2], 4: [1], 6: [2, 3], 7: [3],
        10: [4], 12: [5], 14: [6, 7], 15: [7], 17: [8], 18: [5, 6], 19: [7],
        21: [8], 22: [7, 8], 24: [8], 25: [6], 27: [4], 28: [3], 30: [2],
        31: [1], 33: [0]}

NW = 32
BLK = 16
CHUNK = 32


def _splat(v):
    return jnp.full((BLK,), v, dtype=jnp.int32)


def _tec_kernel(x_hbm, w_hbm, out_hbm, x_v, o_v, w_v):
    wid = lax.axis_index("s") * 2 + lax.axis_index("c")
    per_w = out_hbm.shape[0] // NW
    base_w = wid * per_w

    pltpu.sync_copy(w_hbm, w_v)

    zero = jnp.zeros((BLK,), jnp.float32)
    zero_cols = [p * OUT_L + l for p in range(N_POCKETS) if p not in _P2J
                 for l in range(OUT_L)]
    for blk in range(CHUNK // BLK):
        row = jnp.arange(BLK, dtype=jnp.int32) + blk * BLK
        for c in zero_cols:
            plsc.store_scatter(o_v, [row, _splat(c)], zero)

    def block_body(blk, carry):
        row = jnp.arange(BLK, dtype=jnp.int32) + blk * BLK
        for p, js in _P2J.items():
            encf = plsc.load_gather(x_v, [row, _splat(POCKET_OFF + p)])
            e3 = encf.astype(jnp.int32) * FILTER
            f = [plsc.load_gather(w_v, [e3 + t]) for t in range(FILTER)]
            a = []
            for k in range(AA_REP):
                v = plsc.load_gather(x_v, [row, _splat(1 + js[0] * AA_REP + k)])
                for j in js[1:]:
                    v = v + plsc.load_gather(x_v, [row, _splat(1 + j * AA_REP + k)])
                a.append(v)
            for l in range(OUT_L):
                acc = None
                for t in range(FILTER):
                    k = l - t
                    if 0 <= k < AA_REP:
                        term = f[t] * a[k]
                        acc = term if acc is None else acc + term
                plsc.store_scatter(o_v, [row, _splat(p * OUT_L + l)], acc)
        return carry

    def chunk_body(ci, carry):
        base = base_w + ci * CHUNK
        pltpu.sync_copy(x_hbm.at[pl.ds(base, CHUNK)], x_v)
        lax.fori_loop(0, CHUNK // BLK, block_body, 0)
        pltpu.sync_copy(o_v, out_hbm.at[pl.ds(base, CHUNK)])
        return carry

    lax.fori_loop(0, per_w // CHUNK, chunk_body, 0)


@jax.jit
def kernel(x, W):
    B = x.shape[0]
    w_pad = jnp.zeros((64,), jnp.float32).at[:AA_REP * FILTER].set(W.reshape(-1))

    mesh = plsc.VectorSubcoreMesh(core_axis_name="c", subcore_axis_name="s")
    run = functools.partial(
        pl.kernel,
        mesh=mesh,
        compiler_params=pltpu.CompilerParams(use_tc_tiling_on_sc=False,
                                              needs_layout_passes=False),
        out_type=jax.ShapeDtypeStruct((B, O_COLS), jnp.float32),
        scratch_types=[
            pltpu.VMEM((CHUNK, X_COLS), jnp.float32),
            pltpu.VMEM((CHUNK, O_COLS), jnp.float32),
            pltpu.VMEM((64,), jnp.float32),
        ],
    )(_tec_kernel)
    out = run(x, w_pad)
    return out.reshape(B, N_POCKETS, OUT_L)
